# baseline (speedup 1.0000x reference)
import jax
import jax.numpy as jnp
from jax.experimental import pallas as pl
from jax.experimental.pallas import tpu as pltpu

_B, _DK, _DV = 2048, 256, 256
_BB = 16
_N = _B // _BB


def _body(state_ref, q_ref, k_ref, dout_ref, y_ref, dm_ref):
    p = pl.program_id(0)

    @pl.when(p == 0)
    def _():
        s = state_ref[...]
        q = q_ref[...]
        y_ref[...] = jnp.sum(s * q[:, None, :], axis=-1)

    @pl.when(p == 1)
    def _():
        dm_ref[...] = dout_ref[...][:, :, None] * k_ref[...][:, None, :]


def kernel(state, query, key, d_out, *, interpret=False):
    def _read3(p, j):
        return (j * (1 - p) + (_N - 1) * p, 0, 0)

    def _read2(p, j):
        return (j * (1 - p) + (_N - 1) * p, 0)

    def _write2(p, j):
        return (j * p, 0)

    def _write3(p, j):
        return (j * p, 0, 0)

    y, dm = pl.pallas_call(
        _body,
        grid=(2, _N),
        in_specs=[
            pl.BlockSpec((_BB, _DV, _DK), _read3),
            pl.BlockSpec((_BB, _DK), _read2),
            pl.BlockSpec((_BB, _DK), _write2),
            pl.BlockSpec((_BB, _DV), _write2),
        ],
        out_specs=[
            pl.BlockSpec((_BB, _DV), _read2),
            pl.BlockSpec((_BB, _DV, _DK), _write3),
        ],
        out_shape=[
            jax.ShapeDtypeStruct((_B, _DV), jnp.float32),
            jax.ShapeDtypeStruct((_B, _DV, _DK), jnp.float32),
        ],
        compiler_params=pltpu.CompilerParams(
            dimension_semantics=("arbitrary", "arbitrary"),
            vmem_limit_bytes=48 * 1024 * 1024,
        ),
        name="matrix_memory",
        interpret=interpret,
    )(state, query, key, d_out)
    return (y, dm)


# speedup vs baseline: 1.1250x; 1.1250x over previous
import jax
import jax.numpy as jnp
from jax.experimental import pallas as pl
from jax.experimental.pallas import tpu as pltpu

_B, _DK, _DV = 2048, 256, 256
_BBY = 64
_BBD = 64


def _y_body(state_ref, q_ref, y_ref):
    s = state_ref[...]
    q = q_ref[...]
    y_ref[...] = jnp.sum(s * q[:, None, :], axis=-1)


def _dm_body(k_ref, dout_ref, dm_ref):
    dm_ref[...] = dout_ref[...][:, :, None] * k_ref[...][:, None, :]


def kernel(state, query, key, d_out, *, interpret=False):
    y = pl.pallas_call(
        _y_body,
        grid=(_B // _BBY,),
        in_specs=[
            pl.BlockSpec((_BBY, _DV, _DK), lambda j: (j, 0, 0)),
            pl.BlockSpec((_BBY, _DK), lambda j: (j, 0)),
        ],
        out_specs=pl.BlockSpec((_BBY, _DV), lambda j: (j, 0)),
        out_shape=jax.ShapeDtypeStruct((_B, _DV), jnp.float32),
        compiler_params=pltpu.CompilerParams(
            dimension_semantics=("arbitrary",),
            vmem_limit_bytes=48 * 1024 * 1024,
        ),
        name="mm_read_y",
        interpret=interpret,
    )(state, query)
    dm = pl.pallas_call(
        _dm_body,
        grid=(_B // _BBD,),
        in_specs=[
            pl.BlockSpec((_BBD, _DK), lambda j: (j, 0)),
            pl.BlockSpec((_BBD, _DV), lambda j: (j, 0)),
        ],
        out_specs=pl.BlockSpec((_BBD, _DV, _DK), lambda j: (j, 0, 0)),
        out_shape=jax.ShapeDtypeStruct((_B, _DV, _DK), jnp.float32),
        compiler_params=pltpu.CompilerParams(
            dimension_semantics=("arbitrary",),
            vmem_limit_bytes=48 * 1024 * 1024,
        ),
        name="mm_write_dm",
        interpret=interpret,
    )(key, d_out)
    return (y, dm)


# speedup vs baseline: 1.1301x; 1.0045x over previous
import jax
import jax.numpy as jnp
from jax.experimental import pallas as pl
from jax.experimental.pallas import tpu as pltpu

_B, _DK, _DV = 2048, 256, 256
_BB = 32
_N = _B // _BB


def _body(state_ref, q_ref, k_ref, dout_ref, y_ref, dm_ref):
    p = pl.program_id(0)

    @pl.when(p == 0)
    def _():
        s = state_ref[...]
        q = q_ref[...]
        y_ref[...] = jnp.sum(s * q[:, None, :], axis=-1)

    @pl.when(p == 1)
    def _():
        dm_ref[...] = dout_ref[...][:, :, None] * k_ref[...][:, None, :]


def kernel(state, query, key, d_out, *, interpret=False):
    def _read3(p, j):
        return (j * (1 - p) + (_N - 1) * p, 0, 0)

    def _read2(p, j):
        return (j * (1 - p) + (_N - 1) * p, 0)

    def _write2(p, j):
        return (j * p, 0)

    def _write3(p, j):
        return (j * p, 0, 0)

    y, dm = pl.pallas_call(
        _body,
        grid=(2, _N),
        in_specs=[
            pl.BlockSpec((_BB, _DV, _DK), _read3),
            pl.BlockSpec((_BB, _DK), _read2),
            pl.BlockSpec((_BB, _DK), _write2),
            pl.BlockSpec((_BB, _DV), _write2),
        ],
        out_specs=[
            pl.BlockSpec((_BB, _DV), _read2),
            pl.BlockSpec((_BB, _DV, _DK), _write3),
        ],
        out_shape=[
            jax.ShapeDtypeStruct((_B, _DV), jnp.float32),
            jax.ShapeDtypeStruct((_B, _DV, _DK), jnp.float32),
        ],
        compiler_params=pltpu.CompilerParams(
            dimension_semantics=("arbitrary", "arbitrary"),
            vmem_limit_bytes=48 * 1024 * 1024,
        ),
        name="matrix_memory",
        interpret=interpret,
    )(state, query, key, d_out)
    return (y, dm)


# speedup vs baseline: 1.1318x; 1.0015x over previous
import jax
import jax.numpy as jnp
from jax.experimental import pallas as pl
from jax.experimental.pallas import tpu as pltpu

_B, _DK, _DV = 2048, 256, 256
_BB = 32
_N = _B // _BB


def _body(state_ref, q_ref, k_ref, dout_ref, y_ref, dm_ref):
    p = pl.program_id(0)

    @pl.when(p == 0)
    def _():
        s = state_ref[...]
        q = q_ref[...]
        y_ref[...] = jnp.sum(s * q[:, None, :], axis=-1)

    @pl.when(p == 1)
    def _():
        dm_ref[...] = dout_ref[...][:, :, None] * k_ref[...][:, None, :]


def kernel(state, query, key, d_out):
    def _read3(p, j):
        return (j * (1 - p) + (_N - 1) * p, 0, 0)

    def _read2(p, j):
        return (j * (1 - p) + (_N - 1) * p, 0)

    def _write2(p, j):
        return (j * p, 0)

    def _write3(p, j):
        return (j * p, 0, 0)

    y, dm = pl.pallas_call(
        _body,
        grid=(2, _N),
        in_specs=[
            pl.BlockSpec((_BB, _DV, _DK), _read3),
            pl.BlockSpec((_BB, _DK), _read2),
            pl.BlockSpec((_BB, _DK), _write2),
            pl.BlockSpec((_BB, _DV), _write2),
        ],
        out_specs=[
            pl.BlockSpec((_BB, _DV), _read2),
            pl.BlockSpec((_BB, _DV, _DK), _write3),
        ],
        out_shape=[
            jax.ShapeDtypeStruct((_B, _DV), jnp.float32),
            jax.ShapeDtypeStruct((_B, _DV, _DK), jnp.float32),
        ],
        compiler_params=pltpu.CompilerParams(
            dimension_semantics=("arbitrary", "arbitrary"),
            vmem_limit_bytes=48 * 1024 * 1024,
        ),
        name="matrix_memory",
    )(state, query, key, d_out)
    return (y, dm)
